# baseline (speedup 1.0000x reference)
import functools

import jax
import jax.numpy as jnp
from jax import lax
from jax.experimental import pallas as pl
from jax.experimental.pallas import tpu as pltpu
from jax.experimental.pallas import tpu_sc as plsc

_M = 16384
_BM = 512
_NREF = 2048
_NKEYS = 1024
_D = 128
_EPS = 1e-8


def _topk_body(bpadT_ref, b2_ref, c0_ref, c1_ref, c2_ref, *, off):
    i = pl.program_id(0) + off
    f32 = jnp.float32

    mcol = i * _BM + jax.lax.broadcasted_iota(jnp.int32, (_BM, 1), 0)
    x = (mcol // 128).astype(f32) / f32(127.0)
    y = (mcol % 128).astype(f32) / f32(127.0)
    a2 = x * x + y * y

    lane128 = jax.lax.broadcasted_iota(jnp.int32, (_BM, _D), 1)
    xb = jnp.broadcast_to(x, (_BM, _D))
    yb = jnp.broadcast_to(y, (_BM, _D))
    apad = jnp.where(lane128 == 0, xb, jnp.where(lane128 == 1, yb, f32(0.0)))

    dot = jnp.dot(apad, bpadT_ref[...], preferred_element_type=f32)
    d2 = (a2 + b2_ref[...]) - f32(2.0) * dot
    dist = jnp.sqrt(jnp.maximum(d2, f32(0.0)))

    lane_f = jax.lax.broadcasted_iota(jnp.int32, (_BM, _NREF), 1).astype(f32)
    for c_ref in (c0_ref, c1_ref, c2_ref):
        mn = jnp.min(dist, axis=1, keepdims=True)
        cand = jnp.where(dist == mn, lane_f, f32(_NREF))
        cf = jnp.min(cand, axis=1, keepdims=True)
        dist = jnp.where(lane_f == cf, jnp.inf, dist)
        c_ref[...] = cf.astype(jnp.int32)


def _sim_body(keys_ref, mem_keys_ref, mem_tokens_ref, out_ref, bn_ref):
    f32 = jnp.float32

    @pl.when(pl.program_id(0) == 0)
    def _():
        mk3 = mem_keys_ref[...]
        mkf = jnp.concatenate([mk3[:, 0, :], mk3[:, 1, :], mk3[:, 2, :]],
                              axis=1)
        nrm = jnp.sqrt(jnp.sum(mkf * mkf, axis=1, keepdims=True))
        bn_ref[...] = mkf / jnp.maximum(nrm, f32(_EPS))

    kf = keys_ref[...]

    ssq = jnp.sum(kf * kf, axis=1, keepdims=True)
    an = kf / jnp.maximum(jnp.sqrt(ssq), f32(_EPS))
    sim = jax.lax.dot_general(an, bn_ref[...],
                              dimension_numbers=(((1,), (1,)), ((), ())),
                              preferred_element_type=f32)

    lane_k = jax.lax.broadcasted_iota(jnp.int32, (_BM, _NKEYS), 1)
    lane_kf = lane_k.astype(f32)
    mx = jnp.max(sim, axis=1, keepdims=True)
    candk = jnp.where(sim == mx, lane_kf, f32(_NKEYS))
    ridx = jnp.min(candk, axis=1, keepdims=True)

    ohr = (lane_kf == ridx).astype(f32)
    out_ref[...] = jnp.dot(ohr, mem_tokens_ref[...], preferred_element_type=f32)


def _sc_gather_keys(tokens, c0, c1, c2):
    m = c0.shape[0]
    info = plsc.get_sparse_core_info()
    nc, ns = info.num_cores, info.num_subcores
    nw = nc * ns
    b_per_w = m // nw
    chunk = min(b_per_w, 256)
    nch = b_per_w // chunk
    mesh = plsc.VectorSubcoreMesh(core_axis_name="c", subcore_axis_name="s")

    @functools.partial(
        pl.kernel,
        out_type=jax.ShapeDtypeStruct((m, 3 * _D), jnp.float32),
        mesh=mesh,
        scratch_types=[
            pltpu.VMEM((chunk,), jnp.int32),
            pltpu.VMEM((chunk,), jnp.int32),
            pltpu.VMEM((chunk, _D), jnp.float32),
            pltpu.VMEM((chunk, _D), jnp.float32),
            pltpu.SemaphoreType.DMA,
            pltpu.SemaphoreType.DMA,
            pltpu.SemaphoreType.DMA,
            pltpu.SemaphoreType.DMA,
        ],
    )
    def g(tok_hbm, c0_hbm, c1_hbm, c2_hbm, out_hbm, i0, i1, r0, r1,
          g0, g1, w0, w1):
        wid = lax.axis_index("s") * nc + lax.axis_index("c")
        base = wid * b_per_w
        idx_refs = (c0_hbm, c1_hbm, c2_hbm)
        idx_bufs, row_bufs = (i0, i1), (r0, r1)
        gsems, wsems = (g0, g1), (w0, w1)
        units = [(s, ch) for s in range(3) for ch in range(nch)]
        gather_h = [None, None]
        wb_h = [None, None]
        prev_dst = [None, None]
        for u, (s, ch) in enumerate(units):
            bb = u % 2
            if wb_h[bb] is not None:
                wb_h[bb].wait()
                wb_h[bb] = None
            o = base + ch * chunk
            pltpu.sync_copy(idx_refs[s].at[pl.ds(o, chunk)], idx_bufs[bb])
            gather_h[bb] = pltpu.async_copy(tok_hbm.at[idx_bufs[bb]],
                                            row_bufs[bb], gsems[bb])
            prev_dst[bb] = out_hbm.at[pl.ds(o, chunk), pl.ds(s * _D, _D)]
            pb = 1 - bb
            if gather_h[pb] is not None:
                gather_h[pb].wait()
                gather_h[pb] = None
                wb_h[pb] = pltpu.async_copy(row_bufs[pb], prev_dst[pb],
                                            wsems[pb])
        lb = (len(units) - 1) % 2
        gather_h[lb].wait()
        wb_h[lb] = pltpu.async_copy(row_bufs[lb], prev_dst[lb], wsems[lb])
        for bb in (0, 1):
            if wb_h[bb] is not None:
                wb_h[bb].wait()

    return g(tokens, c0, c1, c2)


def kernel(incomplete_mask, tokens, ref_pts, mem_keys, mem_tokens):
    del incomplete_mask
    f32 = jnp.float32
    bpadT = jnp.pad(ref_pts.T.astype(f32), ((0, _D - 3), (0, 0)))
    b2 = jnp.sum(ref_pts * ref_pts, axis=1)[None, :]

    nhalf = 2
    mh = _M // nhalf
    grid = (mh // _BM,)
    cspec = pl.BlockSpec((_BM, 1), lambda i: (i, 0))

    def topk_half(h):
        cs2d = pl.pallas_call(
            functools.partial(_topk_body, off=h * (mh // _BM)),
            grid=grid,
            in_specs=[
                pl.BlockSpec((_D, _NREF), lambda i: (0, 0)),
                pl.BlockSpec((1, _NREF), lambda i: (0, 0)),
            ],
            out_specs=[cspec, cspec, cspec],
            out_shape=[jax.ShapeDtypeStruct((mh, 1), jnp.int32)] * 3,
        )(bpadT, b2)
        return [c.reshape(mh) for c in cs2d]

    def sim_half(keys_h):
        return pl.pallas_call(
            _sim_body,
            grid=grid,
            in_specs=[
                pl.BlockSpec((_BM, 3 * _D), lambda i: (i, 0)),
                pl.BlockSpec((_NKEYS, 3, _D), lambda i: (0, 0, 0)),
                pl.BlockSpec((_NKEYS, _D), lambda i: (0, 0)),
            ],
            out_specs=pl.BlockSpec((_BM, _D), lambda i: (i, 0)),
            out_shape=jax.ShapeDtypeStruct((mh, _D), f32),
            scratch_shapes=[pltpu.VMEM((_NKEYS, 3 * _D), f32)],
        )(keys_h, mem_keys, mem_tokens)

    cs = [topk_half(h) for h in range(nhalf)]
    keys = [_sc_gather_keys(tokens, *cs[h]) for h in range(nhalf)]
    outs = [sim_half(keys[h]) for h in range(nhalf)]
    return jnp.concatenate(outs, axis=0)


# speedup vs baseline: 1.0282x; 1.0282x over previous
import functools

import jax
import jax.numpy as jnp
from jax import lax
from jax.experimental import pallas as pl
from jax.experimental.pallas import tpu as pltpu
from jax.experimental.pallas import tpu_sc as plsc

_M = 16384
_BM = 512
_NREF = 2048
_NKEYS = 1024
_D = 128
_EPS = 1e-8


def _topk_body(bpadT_ref, b2_ref, c0_ref, c1_ref, c2_ref, *, off):
    i = pl.program_id(0) + off
    f32 = jnp.float32

    mcol = i * _BM + jax.lax.broadcasted_iota(jnp.int32, (_BM, 1), 0)
    x = (mcol // 128).astype(f32) / f32(127.0)
    y = (mcol % 128).astype(f32) / f32(127.0)
    a2 = x * x + y * y

    lane128 = jax.lax.broadcasted_iota(jnp.int32, (_BM, _D), 1)
    xb = jnp.broadcast_to(x, (_BM, _D))
    yb = jnp.broadcast_to(y, (_BM, _D))
    apad = jnp.where(lane128 == 0, xb, jnp.where(lane128 == 1, yb, f32(0.0)))

    dot = jnp.dot(apad, bpadT_ref[...], preferred_element_type=f32)
    d2 = (a2 + b2_ref[...]) - f32(2.0) * dot
    dist = jnp.sqrt(jnp.maximum(d2, f32(0.0)))

    lane_f = jax.lax.broadcasted_iota(jnp.int32, (_BM, _NREF), 1).astype(f32)
    for c_ref in (c0_ref, c1_ref, c2_ref):
        mn = jnp.min(dist, axis=1, keepdims=True)
        cand = jnp.where(dist == mn, lane_f, f32(_NREF))
        cf = jnp.min(cand, axis=1, keepdims=True)
        dist = jnp.where(lane_f == cf, jnp.inf, dist)
        c_ref[...] = cf[:, 0].astype(jnp.int32)


def _sim_body(keys_ref, mem_keys_ref, mem_tokens_ref, out_ref, bn_ref):
    f32 = jnp.float32

    @pl.when(pl.program_id(0) == 0)
    def _():
        mk3 = mem_keys_ref[...]
        mkf = jnp.concatenate([mk3[:, 0, :], mk3[:, 1, :], mk3[:, 2, :]],
                              axis=1)
        nrm = jnp.sqrt(jnp.sum(mkf * mkf, axis=1, keepdims=True))
        bn_ref[...] = mkf / jnp.maximum(nrm, f32(_EPS))

    k3 = keys_ref[...]
    kf = jnp.concatenate([k3[0], k3[1], k3[2]], axis=1)

    ssq = jnp.sum(kf * kf, axis=1, keepdims=True)
    an = kf / jnp.maximum(jnp.sqrt(ssq), f32(_EPS))
    sim = jax.lax.dot_general(an, bn_ref[...],
                              dimension_numbers=(((1,), (1,)), ((), ())),
                              preferred_element_type=f32)

    lane_k = jax.lax.broadcasted_iota(jnp.int32, (_BM, _NKEYS), 1)
    lane_kf = lane_k.astype(f32)
    mx = jnp.max(sim, axis=1, keepdims=True)
    candk = jnp.where(sim == mx, lane_kf, f32(_NKEYS))
    ridx = jnp.min(candk, axis=1, keepdims=True)

    ohr = (lane_kf == ridx).astype(f32)
    out_ref[...] = jnp.dot(ohr, mem_tokens_ref[...], preferred_element_type=f32)


def _sc_gather_keys(tokens, c0, c1, c2):
    m = c0.shape[0]
    info = plsc.get_sparse_core_info()
    nc, ns = info.num_cores, info.num_subcores
    nw = nc * ns
    b_per_w = m // nw
    chunk = min(b_per_w, 256)
    nch = b_per_w // chunk
    mesh = plsc.VectorSubcoreMesh(core_axis_name="c", subcore_axis_name="s")

    @functools.partial(
        pl.kernel,
        out_type=jax.ShapeDtypeStruct((3, m, _D), jnp.float32),
        mesh=mesh,
        scratch_types=[
            pltpu.VMEM((chunk,), jnp.int32),
            pltpu.VMEM((chunk,), jnp.int32),
            pltpu.VMEM((chunk, _D), jnp.float32),
            pltpu.VMEM((chunk, _D), jnp.float32),
            pltpu.SemaphoreType.DMA,
            pltpu.SemaphoreType.DMA,
            pltpu.SemaphoreType.DMA,
            pltpu.SemaphoreType.DMA,
        ],
    )
    def g(tok_hbm, c0_hbm, c1_hbm, c2_hbm, out_hbm, i0, i1, r0, r1,
          g0, g1, w0, w1):
        wid = lax.axis_index("s") * nc + lax.axis_index("c")
        base = wid * b_per_w
        idx_refs = (c0_hbm, c1_hbm, c2_hbm)
        idx_bufs, row_bufs = (i0, i1), (r0, r1)
        gsems, wsems = (g0, g1), (w0, w1)
        units = [(s, ch) for s in range(3) for ch in range(nch)]
        gather_h = [None, None]
        wb_h = [None, None]
        prev_dst = [None, None]
        for u, (s, ch) in enumerate(units):
            bb = u % 2
            if wb_h[bb] is not None:
                wb_h[bb].wait()
                wb_h[bb] = None
            o = base + ch * chunk
            pltpu.sync_copy(idx_refs[s].at[pl.ds(o, chunk)], idx_bufs[bb])
            gather_h[bb] = pltpu.async_copy(tok_hbm.at[idx_bufs[bb]],
                                            row_bufs[bb], gsems[bb])
            prev_dst[bb] = out_hbm.at[s, pl.ds(o, chunk)]
            pb = 1 - bb
            if gather_h[pb] is not None:
                gather_h[pb].wait()
                gather_h[pb] = None
                wb_h[pb] = pltpu.async_copy(row_bufs[pb], prev_dst[pb],
                                            wsems[pb])
        lb = (len(units) - 1) % 2
        gather_h[lb].wait()
        wb_h[lb] = pltpu.async_copy(row_bufs[lb], prev_dst[lb], wsems[lb])
        for bb in (0, 1):
            if wb_h[bb] is not None:
                wb_h[bb].wait()

    return g(tokens, c0, c1, c2)


def kernel(incomplete_mask, tokens, ref_pts, mem_keys, mem_tokens):
    del incomplete_mask
    f32 = jnp.float32
    bpadT = jnp.pad(ref_pts.T.astype(f32), ((0, _D - 3), (0, 0)))
    b2 = jnp.sum(ref_pts * ref_pts, axis=1)[None, :]

    nhalf = 2
    mh = _M // nhalf
    grid = (mh // _BM,)
    cspec = pl.BlockSpec((_BM,), lambda i: (i,))

    def topk_half(h):
        return pl.pallas_call(
            functools.partial(_topk_body, off=h * (mh // _BM)),
            grid=grid,
            in_specs=[
                pl.BlockSpec((_D, _NREF), lambda i: (0, 0)),
                pl.BlockSpec((1, _NREF), lambda i: (0, 0)),
            ],
            out_specs=[cspec, cspec, cspec],
            out_shape=[jax.ShapeDtypeStruct((mh,), jnp.int32)] * 3,
        )(bpadT, b2)

    def sim_half(keys_h):
        return pl.pallas_call(
            _sim_body,
            grid=grid,
            in_specs=[
                pl.BlockSpec((3, _BM, _D), lambda i: (0, i, 0)),
                pl.BlockSpec((_NKEYS, 3, _D), lambda i: (0, 0, 0)),
                pl.BlockSpec((_NKEYS, _D), lambda i: (0, 0)),
            ],
            out_specs=pl.BlockSpec((_BM, _D), lambda i: (i, 0)),
            out_shape=jax.ShapeDtypeStruct((mh, _D), f32),
            scratch_shapes=[pltpu.VMEM((_NKEYS, 3 * _D), f32)],
        )(keys_h, mem_keys, mem_tokens)

    cs = [topk_half(h) for h in range(nhalf)]
    keys = [_sc_gather_keys(tokens, *cs[h]) for h in range(nhalf)]
    outs = [sim_half(keys[h]) for h in range(nhalf)]
    return jnp.concatenate(outs, axis=0)


# speedup vs baseline: 1.0504x; 1.0216x over previous
import functools

import jax
import jax.numpy as jnp
from jax import lax
from jax.experimental import pallas as pl
from jax.experimental.pallas import tpu as pltpu
from jax.experimental.pallas import tpu_sc as plsc

_M = 16384
_BM = 512
_BMA = 1024
_NREF = 2048
_NKEYS = 1024
_D = 128
_EPS = 1e-8


def _topk_body(bpadT_ref, b2_ref, c0_ref, c1_ref, c2_ref, *, off):
    i = pl.program_id(0) + off
    f32 = jnp.float32

    mcol = i * _BMA + jax.lax.broadcasted_iota(jnp.int32, (_BMA, 1), 0)
    x = (mcol // 128).astype(f32) / f32(127.0)
    y = (mcol % 128).astype(f32) / f32(127.0)
    a2 = x * x + y * y

    lane128 = jax.lax.broadcasted_iota(jnp.int32, (_BMA, _D), 1)
    xb = jnp.broadcast_to(x, (_BMA, _D))
    yb = jnp.broadcast_to(y, (_BMA, _D))
    apad = jnp.where(lane128 == 0, xb, jnp.where(lane128 == 1, yb, f32(0.0)))

    dot = jnp.dot(apad, bpadT_ref[...], preferred_element_type=f32)
    d2 = (a2 + b2_ref[...]) - f32(2.0) * dot
    dist = jnp.sqrt(jnp.maximum(d2, f32(0.0)))

    lane_f = jax.lax.broadcasted_iota(jnp.int32, (_BMA, _NREF), 1).astype(f32)
    for c_ref in (c0_ref, c1_ref, c2_ref):
        mn = jnp.min(dist, axis=1, keepdims=True)
        cand = jnp.where(dist == mn, lane_f, f32(_NREF))
        cf = jnp.min(cand, axis=1, keepdims=True)
        dist = jnp.where(lane_f == cf, jnp.inf, dist)
        c_ref[...] = cf[:, 0].astype(jnp.int32)


def _sim_body(keys_ref, mem_keys_ref, mem_tokens_ref, out_ref, bn_ref):
    f32 = jnp.float32

    @pl.when(pl.program_id(0) == 0)
    def _():
        mk3 = mem_keys_ref[...]
        mkf = jnp.concatenate([mk3[:, 0, :], mk3[:, 1, :], mk3[:, 2, :]],
                              axis=1)
        nrm = jnp.sqrt(jnp.sum(mkf * mkf, axis=1, keepdims=True))
        bn_ref[...] = mkf / jnp.maximum(nrm, f32(_EPS))

    k3 = keys_ref[...]
    kf = jnp.concatenate([k3[0], k3[1], k3[2]], axis=1)

    ssq = jnp.sum(kf * kf, axis=1, keepdims=True)
    an = kf / jnp.maximum(jnp.sqrt(ssq), f32(_EPS))
    sim = jax.lax.dot_general(an, bn_ref[...],
                              dimension_numbers=(((1,), (1,)), ((), ())),
                              preferred_element_type=f32)

    lane_k = jax.lax.broadcasted_iota(jnp.int32, (_BM, _NKEYS), 1)
    lane_kf = lane_k.astype(f32)
    mx = jnp.max(sim, axis=1, keepdims=True)
    candk = jnp.where(sim == mx, lane_kf, f32(_NKEYS))
    ridx = jnp.min(candk, axis=1, keepdims=True)

    ohr = (lane_kf == ridx).astype(f32)
    out_ref[...] = jnp.dot(ohr, mem_tokens_ref[...], preferred_element_type=f32)


def _sc_gather_keys(tokens, c0, c1, c2):
    m = c0.shape[0]
    info = plsc.get_sparse_core_info()
    nc, ns = info.num_cores, info.num_subcores
    nw = nc * ns
    b_per_w = m // nw
    chunk = min(b_per_w, 256)
    nch = b_per_w // chunk
    mesh = plsc.VectorSubcoreMesh(core_axis_name="c", subcore_axis_name="s")

    @functools.partial(
        pl.kernel,
        out_type=jax.ShapeDtypeStruct((3, m, _D), jnp.float32),
        mesh=mesh,
        scratch_types=[
            pltpu.VMEM((chunk,), jnp.int32),
            pltpu.VMEM((chunk,), jnp.int32),
            pltpu.VMEM((chunk, _D), jnp.float32),
            pltpu.VMEM((chunk, _D), jnp.float32),
            pltpu.SemaphoreType.DMA,
            pltpu.SemaphoreType.DMA,
            pltpu.SemaphoreType.DMA,
            pltpu.SemaphoreType.DMA,
        ],
    )
    def g(tok_hbm, c0_hbm, c1_hbm, c2_hbm, out_hbm, i0, i1, r0, r1,
          g0, g1, w0, w1):
        wid = lax.axis_index("s") * nc + lax.axis_index("c")
        base = wid * b_per_w
        idx_refs = (c0_hbm, c1_hbm, c2_hbm)
        idx_bufs, row_bufs = (i0, i1), (r0, r1)
        gsems, wsems = (g0, g1), (w0, w1)
        units = [(s, ch) for s in range(3) for ch in range(nch)]
        gather_h = [None, None]
        wb_h = [None, None]
        prev_dst = [None, None]
        for u, (s, ch) in enumerate(units):
            bb = u % 2
            if wb_h[bb] is not None:
                wb_h[bb].wait()
                wb_h[bb] = None
            o = base + ch * chunk
            pltpu.sync_copy(idx_refs[s].at[pl.ds(o, chunk)], idx_bufs[bb])
            gather_h[bb] = pltpu.async_copy(tok_hbm.at[idx_bufs[bb]],
                                            row_bufs[bb], gsems[bb])
            prev_dst[bb] = out_hbm.at[s, pl.ds(o, chunk)]
            pb = 1 - bb
            if gather_h[pb] is not None:
                gather_h[pb].wait()
                gather_h[pb] = None
                wb_h[pb] = pltpu.async_copy(row_bufs[pb], prev_dst[pb],
                                            wsems[pb])
        lb = (len(units) - 1) % 2
        gather_h[lb].wait()
        wb_h[lb] = pltpu.async_copy(row_bufs[lb], prev_dst[lb], wsems[lb])
        for bb in (0, 1):
            if wb_h[bb] is not None:
                wb_h[bb].wait()

    return g(tokens, c0, c1, c2)


def kernel(incomplete_mask, tokens, ref_pts, mem_keys, mem_tokens):
    del incomplete_mask
    f32 = jnp.float32
    bpadT = jnp.pad(ref_pts.T.astype(f32), ((0, _D - 3), (0, 0)))
    b2 = jnp.sum(ref_pts * ref_pts, axis=1)[None, :]

    nhalf = 2
    mh = _M // nhalf
    grid = (mh // _BM,)
    grid_a = (mh // _BMA,)
    cspec = pl.BlockSpec((_BMA,), lambda i: (i,))

    def topk_half(h):
        return pl.pallas_call(
            functools.partial(_topk_body, off=h * (mh // _BMA)),
            grid=grid_a,
            in_specs=[
                pl.BlockSpec((_D, _NREF), lambda i: (0, 0)),
                pl.BlockSpec((1, _NREF), lambda i: (0, 0)),
            ],
            out_specs=[cspec, cspec, cspec],
            out_shape=[jax.ShapeDtypeStruct((mh,), jnp.int32)] * 3,
        )(bpadT, b2)

    def sim_half(keys_h):
        return pl.pallas_call(
            _sim_body,
            grid=grid,
            in_specs=[
                pl.BlockSpec((3, _BM, _D), lambda i: (0, i, 0)),
                pl.BlockSpec((_NKEYS, 3, _D), lambda i: (0, 0, 0)),
                pl.BlockSpec((_NKEYS, _D), lambda i: (0, 0)),
            ],
            out_specs=pl.BlockSpec((_BM, _D), lambda i: (i, 0)),
            out_shape=jax.ShapeDtypeStruct((mh, _D), f32),
            scratch_shapes=[pltpu.VMEM((_NKEYS, 3 * _D), f32)],
        )(keys_h, mem_keys, mem_tokens)

    cs = [topk_half(h) for h in range(nhalf)]
    keys = [_sc_gather_keys(tokens, *cs[h]) for h in range(nhalf)]
    outs = [sim_half(keys[h]) for h in range(nhalf)]
    return jnp.concatenate(outs, axis=0)


# speedup vs baseline: 1.0517x; 1.0013x over previous
import functools

import jax
import jax.numpy as jnp
from jax import lax
from jax.experimental import pallas as pl
from jax.experimental.pallas import tpu as pltpu
from jax.experimental.pallas import tpu_sc as plsc

_M = 16384
_BM = 512
_BMA = 1024
_NREF = 2048
_NKEYS = 1024
_D = 128
_EPS = 1e-8


def _topk_body(bpadT_ref, b2_ref, c0_ref, c1_ref, c2_ref, *, off):
    i = pl.program_id(0) + off
    f32 = jnp.float32

    mcol = i * _BMA + jax.lax.broadcasted_iota(jnp.int32, (_BMA, 1), 0)
    x = (mcol // 128).astype(f32) / f32(127.0)
    y = (mcol % 128).astype(f32) / f32(127.0)
    a2 = x * x + y * y

    lane128 = jax.lax.broadcasted_iota(jnp.int32, (_BMA, _D), 1)
    xb = jnp.broadcast_to(x, (_BMA, _D))
    yb = jnp.broadcast_to(y, (_BMA, _D))
    apad = jnp.where(lane128 == 0, xb, jnp.where(lane128 == 1, yb, f32(0.0)))

    dot = jnp.dot(apad, bpadT_ref[...], preferred_element_type=f32)
    d2 = (a2 + b2_ref[...]) - f32(2.0) * dot
    dist = jnp.sqrt(jnp.maximum(d2, f32(0.0)))

    lane_f = jax.lax.broadcasted_iota(jnp.int32, (_BMA, _NREF), 1).astype(f32)
    for c_ref in (c0_ref, c1_ref, c2_ref):
        mn = jnp.min(dist, axis=1, keepdims=True)
        cand = jnp.where(dist == mn, lane_f, f32(_NREF))
        cf = jnp.min(cand, axis=1, keepdims=True)
        dist = jnp.where(lane_f == cf, jnp.inf, dist)
        c_ref[...] = cf[:, 0].astype(jnp.int32)


def _sim_body(keys_ref, mem_keys_ref, mem_tokens_ref, out_ref, bn_ref):
    f32 = jnp.float32

    @pl.when(pl.program_id(0) == 0)
    def _():
        mk3 = mem_keys_ref[...]
        mkf = jnp.concatenate([mk3[:, 0, :], mk3[:, 1, :], mk3[:, 2, :]],
                              axis=1)
        nrm = jnp.sqrt(jnp.sum(mkf * mkf, axis=1, keepdims=True))
        bn_ref[...] = mkf / jnp.maximum(nrm, f32(_EPS))

    k3 = keys_ref[...]
    kf = jnp.concatenate([k3[0], k3[1], k3[2]], axis=1)

    ssq = jnp.sum(kf * kf, axis=1, keepdims=True)
    an = kf / jnp.maximum(jnp.sqrt(ssq), f32(_EPS))
    sim = jax.lax.dot_general(an, bn_ref[...],
                              dimension_numbers=(((1,), (1,)), ((), ())),
                              preferred_element_type=f32)

    lane_k = jax.lax.broadcasted_iota(jnp.int32, (_BM, _NKEYS), 1)
    lane_kf = lane_k.astype(f32)
    mx = jnp.max(sim, axis=1, keepdims=True)
    candk = jnp.where(sim == mx, lane_kf, f32(_NKEYS))
    ridx = jnp.min(candk, axis=1, keepdims=True)

    ohr = (lane_kf == ridx).astype(f32)
    out_ref[...] = jnp.dot(ohr, mem_tokens_ref[...], preferred_element_type=f32)


def _sc_gather_keys(tokens, c0, c1, c2):
    m = c0.shape[0]
    info = plsc.get_sparse_core_info()
    nc, ns = info.num_cores, info.num_subcores
    nw = nc * ns
    b_per_w = m // nw
    chunk = min(b_per_w, 128)
    nch = b_per_w // chunk
    mesh = plsc.VectorSubcoreMesh(core_axis_name="c", subcore_axis_name="s")

    @functools.partial(
        pl.kernel,
        out_type=jax.ShapeDtypeStruct((3, m, _D), jnp.float32),
        mesh=mesh,
        scratch_types=[
            pltpu.VMEM((chunk,), jnp.int32),
            pltpu.VMEM((chunk,), jnp.int32),
            pltpu.VMEM((chunk, _D), jnp.float32),
            pltpu.VMEM((chunk, _D), jnp.float32),
            pltpu.SemaphoreType.DMA,
            pltpu.SemaphoreType.DMA,
            pltpu.SemaphoreType.DMA,
            pltpu.SemaphoreType.DMA,
        ],
    )
    def g(tok_hbm, c0_hbm, c1_hbm, c2_hbm, out_hbm, i0, i1, r0, r1,
          g0, g1, w0, w1):
        wid = lax.axis_index("s") * nc + lax.axis_index("c")
        base = wid * b_per_w
        idx_refs = (c0_hbm, c1_hbm, c2_hbm)
        idx_bufs, row_bufs = (i0, i1), (r0, r1)
        gsems, wsems = (g0, g1), (w0, w1)
        units = [(s, ch) for s in range(3) for ch in range(nch)]
        gather_h = [None, None]
        wb_h = [None, None]
        prev_dst = [None, None]
        for u, (s, ch) in enumerate(units):
            bb = u % 2
            if wb_h[bb] is not None:
                wb_h[bb].wait()
                wb_h[bb] = None
            o = base + ch * chunk
            pltpu.sync_copy(idx_refs[s].at[pl.ds(o, chunk)], idx_bufs[bb])
            gather_h[bb] = pltpu.async_copy(tok_hbm.at[idx_bufs[bb]],
                                            row_bufs[bb], gsems[bb])
            prev_dst[bb] = out_hbm.at[s, pl.ds(o, chunk)]
            pb = 1 - bb
            if gather_h[pb] is not None:
                gather_h[pb].wait()
                gather_h[pb] = None
                wb_h[pb] = pltpu.async_copy(row_bufs[pb], prev_dst[pb],
                                            wsems[pb])
        lb = (len(units) - 1) % 2
        gather_h[lb].wait()
        wb_h[lb] = pltpu.async_copy(row_bufs[lb], prev_dst[lb], wsems[lb])
        for bb in (0, 1):
            if wb_h[bb] is not None:
                wb_h[bb].wait()

    return g(tokens, c0, c1, c2)


def kernel(incomplete_mask, tokens, ref_pts, mem_keys, mem_tokens):
    del incomplete_mask
    f32 = jnp.float32
    bpadT = jnp.pad(ref_pts.T.astype(f32), ((0, _D - 3), (0, 0)))
    b2 = jnp.sum(ref_pts * ref_pts, axis=1)[None, :]

    nhalf = 2
    mh = _M // nhalf
    grid = (mh // _BM,)
    grid_a = (mh // _BMA,)
    cspec = pl.BlockSpec((_BMA,), lambda i: (i,))

    def topk_half(h):
        return pl.pallas_call(
            functools.partial(_topk_body, off=h * (mh // _BMA)),
            grid=grid_a,
            in_specs=[
                pl.BlockSpec((_D, _NREF), lambda i: (0, 0)),
                pl.BlockSpec((1, _NREF), lambda i: (0, 0)),
            ],
            out_specs=[cspec, cspec, cspec],
            out_shape=[jax.ShapeDtypeStruct((mh,), jnp.int32)] * 3,
        )(bpadT, b2)

    def sim_half(keys_h):
        return pl.pallas_call(
            _sim_body,
            grid=grid,
            in_specs=[
                pl.BlockSpec((3, _BM, _D), lambda i: (0, i, 0)),
                pl.BlockSpec((_NKEYS, 3, _D), lambda i: (0, 0, 0)),
                pl.BlockSpec((_NKEYS, _D), lambda i: (0, 0)),
            ],
            out_specs=pl.BlockSpec((_BM, _D), lambda i: (i, 0)),
            out_shape=jax.ShapeDtypeStruct((mh, _D), f32),
            scratch_shapes=[pltpu.VMEM((_NKEYS, 3 * _D), f32)],
        )(keys_h, mem_keys, mem_tokens)

    cs = [topk_half(h) for h in range(nhalf)]
    keys = [_sc_gather_keys(tokens, *cs[h]) for h in range(nhalf)]
    outs = [sim_half(keys[h]) for h in range(nhalf)]
    return jnp.concatenate(outs, axis=0)
